# Initial kernel scaffold; baseline (speedup 1.0000x reference)
import jax
import jax.numpy as jnp
from jax.experimental import pallas as pl


def kernel(positions, pos_emb, W, b):
    raise NotImplementedError("write your pallas kernel here")


# speedup vs baseline: 4.1308x; 4.1308x over previous
import functools

import jax
import jax.numpy as jnp
from jax import lax
from jax.experimental import pallas as pl
from jax.experimental.pallas import tpu as pltpu
from jax.experimental.pallas import tpu_sc as plsc

_OUT_DIM = 64

_NC = 2
_NS = 16
_NW = _NC * _NS

_CHUNK = 1024


def _table_body(pos_emb_ref, w_ref, b_ref, t_ref):
    t_ref[...] = lax.dot_general(
        pos_emb_ref[...], w_ref[...],
        dimension_numbers=(((1,), (1,)), ((), ())),
        preferred_element_type=jnp.float32,
    ) + b_ref[...]


def _make_table(pos_emb, W, b):
    return pl.pallas_call(
        _table_body,
        out_shape=jax.ShapeDtypeStruct(pos_emb.shape, jnp.float32),
    )(pos_emb, W, b.reshape(1, _OUT_DIM))


def _gather_body(n_per_w, n_chunks, table_hbm, idx_hbm, out_hbm, idx_v, rows_v, sem):
    wid = lax.axis_index("s") * _NC + lax.axis_index("c")
    base = wid * n_per_w

    def step(i, carry):
        off = base + i * _CHUNK
        pltpu.sync_copy(idx_hbm.at[pl.ds(off, _CHUNK)], idx_v)
        pltpu.async_copy(table_hbm.at[idx_v], rows_v, sem).wait()
        pltpu.sync_copy(rows_v, out_hbm.at[pl.ds(off, _CHUNK)])
        return carry

    lax.fori_loop(0, n_chunks, step, 0)


def _make_gather(n_total):
    n_per_w = n_total // _NW
    n_chunks = n_per_w // _CHUNK
    mesh = plsc.VectorSubcoreMesh(core_axis_name="c", subcore_axis_name="s")
    return functools.partial(
        pl.kernel,
        mesh=mesh,
        out_type=jax.ShapeDtypeStruct((n_total, _OUT_DIM), jnp.float32),
        scratch_types=[
            pltpu.VMEM((_CHUNK,), jnp.int32),
            pltpu.VMEM((_CHUNK, _OUT_DIM), jnp.float32),
            pltpu.SemaphoreType.DMA,
        ],
        compiler_params=pltpu.CompilerParams(use_tc_tiling_on_sc=False),
    )(functools.partial(_gather_body, n_per_w, n_chunks))


def kernel(positions, pos_emb, W, b):
    batch, hist = positions.shape
    n_total = batch * hist
    table = _make_table(pos_emb, W, b)
    idx = positions.reshape(n_total).astype(jnp.int32)
    out = _make_gather(n_total)(table, idx)
    return out.reshape(batch, hist, _OUT_DIM)


# speedup vs baseline: 4.1710x; 1.0098x over previous
import functools

import jax
import jax.numpy as jnp
from jax import lax
from jax.experimental import pallas as pl
from jax.experimental.pallas import tpu as pltpu
from jax.experimental.pallas import tpu_sc as plsc

_OUT_DIM = 64

_NC = 2
_NS = 16
_NW = _NC * _NS

_CHUNK = 800


def _table_body(pos_emb_ref, w_ref, b_ref, t_ref):
    t_ref[...] = lax.dot_general(
        pos_emb_ref[...], w_ref[...],
        dimension_numbers=(((1,), (1,)), ((), ())),
        preferred_element_type=jnp.float32,
    ) + b_ref[...]


def _make_table(pos_emb, W, b):
    return pl.pallas_call(
        _table_body,
        out_shape=jax.ShapeDtypeStruct(pos_emb.shape, jnp.float32),
    )(pos_emb, W, b.reshape(1, _OUT_DIM))


def _gather_body(n_per_w, n_chunks, table_hbm, idx_hbm, out_hbm,
                 idx_all, rows0, rows1, sg0, sg1, sw0, sw1):
    wid = lax.axis_index("s") * _NC + lax.axis_index("c")
    base = wid * n_per_w
    pltpu.sync_copy(idx_hbm.at[pl.ds(base, n_per_w)], idx_all)

    def idx_slice(i):
        return idx_all.at[pl.ds(i * _CHUNK, _CHUNK)]

    def out_slice(i):
        return out_hbm.at[pl.ds(base + i * _CHUNK, _CHUNK)]

    pltpu.async_copy(table_hbm.at[idx_slice(0)], rows0, sg0)
    pltpu.async_copy(table_hbm.at[idx_slice(1)], rows1, sg1)

    def pair(j, carry):
        i0 = j * 2
        pltpu.make_async_copy(table_hbm.at[idx_slice(i0)], rows0, sg0).wait()
        pltpu.async_copy(rows0, out_slice(i0), sw0)
        pltpu.make_async_copy(table_hbm.at[idx_slice(i0 + 1)], rows1, sg1).wait()
        pltpu.async_copy(rows1, out_slice(i0 + 1), sw1)

        @pl.when(j < n_chunks // 2 - 1)
        def _():
            pltpu.make_async_copy(rows0, out_slice(i0), sw0).wait()
            pltpu.async_copy(table_hbm.at[idx_slice(i0 + 2)], rows0, sg0)
            pltpu.make_async_copy(rows1, out_slice(i0 + 1), sw1).wait()
            pltpu.async_copy(table_hbm.at[idx_slice(i0 + 3)], rows1, sg1)

        return carry

    lax.fori_loop(0, n_chunks // 2, pair, 0)

    pltpu.make_async_copy(rows0, out_slice(0), sw0).wait()
    pltpu.make_async_copy(rows1, out_slice(1), sw1).wait()


def _make_gather(n_total):
    n_per_w = n_total // _NW
    n_chunks = n_per_w // _CHUNK
    assert n_chunks % 2 == 0
    mesh = plsc.VectorSubcoreMesh(core_axis_name="c", subcore_axis_name="s")
    return functools.partial(
        pl.kernel,
        mesh=mesh,
        out_type=jax.ShapeDtypeStruct((n_total, _OUT_DIM), jnp.float32),
        scratch_types=[
            pltpu.VMEM((n_per_w,), jnp.int32),
            pltpu.VMEM((_CHUNK, _OUT_DIM), jnp.float32),
            pltpu.VMEM((_CHUNK, _OUT_DIM), jnp.float32),
            pltpu.SemaphoreType.DMA,
            pltpu.SemaphoreType.DMA,
            pltpu.SemaphoreType.DMA,
            pltpu.SemaphoreType.DMA,
        ],
        compiler_params=pltpu.CompilerParams(use_tc_tiling_on_sc=False),
    )(functools.partial(_gather_body, n_per_w, n_chunks))


def kernel(positions, pos_emb, W, b):
    batch, hist = positions.shape
    n_total = batch * hist
    table = _make_table(pos_emb, W, b)
    idx = positions.reshape(n_total).astype(jnp.int32)
    out = _make_gather(n_total)(table, idx)
    return out.reshape(batch, hist, _OUT_DIM)
